# baseline (speedup 1.0000x reference)
import functools

import jax
import jax.numpy as jnp
from jax import lax
from jax.experimental import pallas as pl
from jax.experimental.pallas import tpu as pltpu
from jax.experimental.pallas import tpu_sc as plsc

NUM_ENTITIES = 100000
NUM_RELATIONS = 1000
D = 128
B = 16384
L = 16

NC = 2
NS = 16
NW = NC * NS
RPW = B // NW
CHUNK = 64
NCHUNK = RPW // CHUNK
NBUF = 3


def _body(h_hbm, r_hbm, t_hbm, ent_hbm, rel_hbm, out_hbm,
          ihall, irall, itall,
          hrow0, rrow0, trow0, hrow1, rrow1, trow1, hrow2, rrow2, trow2,
          pacc, outbuf, sem0, sem1, sem2):
    wid = lax.axis_index("s") * NC + lax.axis_index("c")
    base = wid * RPW
    lane = lax.iota(jnp.int32, L)
    colbase = lane * L

    icps = (pltpu.make_async_copy(h_hbm.at[pl.ds(base, RPW)], ihall, sem0),
            pltpu.make_async_copy(r_hbm.at[pl.ds(base, RPW)], irall, sem0),
            pltpu.make_async_copy(t_hbm.at[pl.ds(base, RPW)], itall, sem0))
    for cp in icps:
        cp.start()
    for cp in icps:
        cp.wait()

    bufs = [(hrow0, rrow0, trow0), (hrow1, rrow1, trow1), (hrow2, rrow2, trow2)]
    sems = [sem0, sem1, sem2]

    def fire(c):
        hb, rb, tb = bufs[c % NBUF]
        s = sems[c % NBUF]
        sl = pl.ds(c * CHUNK, CHUNK)
        cps = (pltpu.make_async_copy(ent_hbm.at[ihall.at[sl]], hb, s),
               pltpu.make_async_copy(rel_hbm.at[irall.at[sl]], rb, s),
               pltpu.make_async_copy(ent_hbm.at[itall.at[sl]], tb, s))
        for cp in cps:
            cp.start()
        return cps

    def compute(c):
        hb, rb, tb = bufs[c % NBUF]
        off = c * CHUNK

        def group_body(g, _):
            rowbase = g * L
            for j in range(L):
                row = rowbase + j
                m = [hb[row, pl.ds(k * L, L)]
                     * rb[row, pl.ds(k * L, L)]
                     * tb[row, pl.ds(k * L, L)]
                     for k in range(D // L)]
                a0 = m[0] + m[1]
                a1 = m[2] + m[3]
                a2 = m[4] + m[5]
                a3 = m[6] + m[7]
                pacc[pl.ds(j * L, L)] = (a0 + a1) + (a2 + a3)
            s = plsc.load_gather(pacc, [colbase])
            for l in range(1, L):
                s = s + plsc.load_gather(pacc, [colbase + l])
            outbuf[pl.ds(off + rowbase, L)] = s
            return 0

        lax.fori_loop(0, CHUNK // L, group_body, 0)

    inflight = [fire(0), fire(1)]
    for c in range(NCHUNK):
        if c + 2 < NCHUNK:
            inflight.append(fire(c + 2))
        for cp in inflight.pop(0):
            cp.wait()
        compute(c)

    pltpu.sync_copy(outbuf, out_hbm.at[pl.ds(base, RPW)])


def _distmult_sc(h, r, t, ent, rel):
    mesh = plsc.VectorSubcoreMesh(core_axis_name="c", subcore_axis_name="s")
    k = functools.partial(
        pl.kernel,
        out_type=jax.ShapeDtypeStruct((B,), jnp.float32),
        mesh=mesh,
        compiler_params=pltpu.CompilerParams(needs_layout_passes=False),
        scratch_types=[
            pltpu.VMEM((RPW,), jnp.int32),
            pltpu.VMEM((RPW,), jnp.int32),
            pltpu.VMEM((RPW,), jnp.int32),
            pltpu.VMEM((CHUNK, D), jnp.float32),
            pltpu.VMEM((CHUNK, D), jnp.float32),
            pltpu.VMEM((CHUNK, D), jnp.float32),
            pltpu.VMEM((CHUNK, D), jnp.float32),
            pltpu.VMEM((CHUNK, D), jnp.float32),
            pltpu.VMEM((CHUNK, D), jnp.float32),
            pltpu.VMEM((CHUNK, D), jnp.float32),
            pltpu.VMEM((CHUNK, D), jnp.float32),
            pltpu.VMEM((CHUNK, D), jnp.float32),
            pltpu.VMEM((L * L,), jnp.float32),
            pltpu.VMEM((RPW,), jnp.float32),
            pltpu.SemaphoreType.DMA,
            pltpu.SemaphoreType.DMA,
            pltpu.SemaphoreType.DMA,
        ],
    )(_body)
    return k(h, r, t, ent, rel)


def kernel(h, r, t, entity_embeddings, relation_embeddings):
    h = jnp.asarray(h, jnp.int32)
    r = jnp.asarray(r, jnp.int32)
    t = jnp.asarray(t, jnp.int32)
    return _distmult_sc(h, r, t, entity_embeddings, relation_embeddings)


# speedup vs baseline: 1.0795x; 1.0795x over previous
import functools

import jax
import jax.numpy as jnp
from jax import lax
from jax.experimental import pallas as pl
from jax.experimental.pallas import tpu as pltpu
from jax.experimental.pallas import tpu_sc as plsc

NUM_ENTITIES = 100000
NUM_RELATIONS = 1000
D = 128
B = 16384
L = 16

NC = 2
NS = 16
NW = NC * NS
RPW = B // NW
CHUNK = 128
NCHUNK = RPW // CHUNK
NBUF = 2


def _body(h_hbm, r_hbm, t_hbm, ent_hbm, rel_hbm, out_hbm,
          ihall, irall, itall,
          hrow0, rrow0, trow0, hrow1, rrow1, trow1,
          pacc, outbuf, sem0, sem1):
    wid = lax.axis_index("s") * NC + lax.axis_index("c")
    base = wid * RPW
    lane = lax.iota(jnp.int32, L)
    colbase = lane * L

    icps = (pltpu.make_async_copy(h_hbm.at[pl.ds(base, RPW)], ihall, sem0),
            pltpu.make_async_copy(r_hbm.at[pl.ds(base, RPW)], irall, sem0),
            pltpu.make_async_copy(t_hbm.at[pl.ds(base, RPW)], itall, sem0))
    for cp in icps:
        cp.start()
    for cp in icps:
        cp.wait()

    bufs = [(hrow0, rrow0, trow0), (hrow1, rrow1, trow1)]
    sems = [sem0, sem1]

    def fire(c):
        hb, rb, tb = bufs[c % NBUF]
        s = sems[c % NBUF]
        sl = pl.ds(c * CHUNK, CHUNK)
        cps = (pltpu.make_async_copy(ent_hbm.at[ihall.at[sl]], hb, s),
               pltpu.make_async_copy(rel_hbm.at[irall.at[sl]], rb, s),
               pltpu.make_async_copy(ent_hbm.at[itall.at[sl]], tb, s))
        for cp in cps:
            cp.start()
        return cps

    def compute(c):
        hb, rb, tb = bufs[c % NBUF]
        off = c * CHUNK

        def group_body(g, _):
            rowbase = g * L
            for j in range(L):
                row = rowbase + j
                m = [hb[row, pl.ds(k * L, L)]
                     * rb[row, pl.ds(k * L, L)]
                     * tb[row, pl.ds(k * L, L)]
                     for k in range(D // L)]
                a0 = m[0] + m[1]
                a1 = m[2] + m[3]
                a2 = m[4] + m[5]
                a3 = m[6] + m[7]
                pacc[pl.ds(j * L, L)] = (a0 + a1) + (a2 + a3)
            s = plsc.load_gather(pacc, [colbase])
            for l in range(1, L):
                s = s + plsc.load_gather(pacc, [colbase + l])
            outbuf[pl.ds(off + rowbase, L)] = s
            return 0

        lax.fori_loop(0, CHUNK // L, group_body, 0)

    inflight = [fire(0)]
    for c in range(NCHUNK):
        if c + 1 < NCHUNK:
            inflight.append(fire(c + 1))
        for cp in inflight.pop(0):
            cp.wait()
        compute(c)

    pltpu.sync_copy(outbuf, out_hbm.at[pl.ds(base, RPW)])


def _distmult_sc(h, r, t, ent, rel):
    mesh = plsc.VectorSubcoreMesh(core_axis_name="c", subcore_axis_name="s")
    k = functools.partial(
        pl.kernel,
        out_type=jax.ShapeDtypeStruct((B,), jnp.float32),
        mesh=mesh,
        compiler_params=pltpu.CompilerParams(needs_layout_passes=False),
        scratch_types=[
            pltpu.VMEM((RPW,), jnp.int32),
            pltpu.VMEM((RPW,), jnp.int32),
            pltpu.VMEM((RPW,), jnp.int32),
            pltpu.VMEM((CHUNK, D), jnp.float32),
            pltpu.VMEM((CHUNK, D), jnp.float32),
            pltpu.VMEM((CHUNK, D), jnp.float32),
            pltpu.VMEM((CHUNK, D), jnp.float32),
            pltpu.VMEM((CHUNK, D), jnp.float32),
            pltpu.VMEM((CHUNK, D), jnp.float32),
            pltpu.VMEM((L * L,), jnp.float32),
            pltpu.VMEM((RPW,), jnp.float32),
            pltpu.SemaphoreType.DMA,
            pltpu.SemaphoreType.DMA,
        ],
    )(_body)
    return k(h, r, t, ent, rel)


def kernel(h, r, t, entity_embeddings, relation_embeddings):
    h = jnp.asarray(h, jnp.int32)
    r = jnp.asarray(r, jnp.int32)
    t = jnp.asarray(t, jnp.int32)
    return _distmult_sc(h, r, t, entity_embeddings, relation_embeddings)


# speedup vs baseline: 1.1626x; 1.0770x over previous
import functools

import jax
import jax.numpy as jnp
from jax import lax
from jax.experimental import pallas as pl
from jax.experimental.pallas import tpu as pltpu
from jax.experimental.pallas import tpu_sc as plsc

NUM_ENTITIES = 100000
NUM_RELATIONS = 1000
D = 128
B = 16384
L = 16

NC = 2
NS = 16
NW = NC * NS
RPW = B // NW
CHUNK = 128
NCHUNK = RPW // CHUNK
NBUF = 2


def _body(h_hbm, r_hbm, t_hbm, ent_hbm, rel_hbm, out_hbm,
          ihall, irall, itall,
          hrow0, rrow0, trow0, hrow1, rrow1, trow1,
          pacc, outbuf, sem0, sem1):
    wid = lax.axis_index("s") * NC + lax.axis_index("c")
    base = wid * RPW
    lane = lax.iota(jnp.int32, L)
    colbase = lane * L

    icps = (pltpu.make_async_copy(h_hbm.at[pl.ds(base, RPW)], ihall, sem0),
            pltpu.make_async_copy(r_hbm.at[pl.ds(base, RPW)], irall, sem0),
            pltpu.make_async_copy(t_hbm.at[pl.ds(base, RPW)], itall, sem0))
    for cp in icps:
        cp.start()
    for cp in icps:
        cp.wait()

    bufs = [(hrow0, rrow0, trow0), (hrow1, rrow1, trow1)]
    sems = [sem0, sem1]

    def fire(c, b):
        hb, rb, tb = bufs[b]
        s = sems[b]
        sl = pl.ds(c * CHUNK, CHUNK)
        for cp in (pltpu.make_async_copy(ent_hbm.at[ihall.at[sl]], hb, s),
                   pltpu.make_async_copy(rel_hbm.at[irall.at[sl]], rb, s),
                   pltpu.make_async_copy(ent_hbm.at[itall.at[sl]], tb, s)):
            cp.start()

    def wait_buf(b):
        hb, rb, tb = bufs[b]
        s = sems[b]
        sl = pl.ds(0, CHUNK)
        for cp in (pltpu.make_async_copy(ent_hbm.at[ihall.at[sl]], hb, s),
                   pltpu.make_async_copy(rel_hbm.at[irall.at[sl]], rb, s),
                   pltpu.make_async_copy(ent_hbm.at[itall.at[sl]], tb, s)):
            cp.wait()

    def compute(c, b):
        hb, rb, tb = bufs[b]
        off = c * CHUNK

        def group_body(g, _):
            rowbase = g * L
            for j in range(L):
                row = rowbase + j
                m = [hb[row, pl.ds(k * L, L)]
                     * rb[row, pl.ds(k * L, L)]
                     * tb[row, pl.ds(k * L, L)]
                     for k in range(D // L)]
                a0 = m[0] + m[1]
                a1 = m[2] + m[3]
                a2 = m[4] + m[5]
                a3 = m[6] + m[7]
                pacc[pl.ds(j * L, L)] = (a0 + a1) + (a2 + a3)
            s = plsc.load_gather(pacc, [colbase])
            for l in range(1, L):
                s = s + plsc.load_gather(pacc, [colbase + l])
            outbuf[pl.ds(off + rowbase, L)] = s
            return 0

        lax.fori_loop(0, CHUNK // L, group_body, 0)

    fire(0, 0)
    fire(1, 1)

    def pair_body(i, _):
        c0 = 2 * i
        wait_buf(0)

        @pl.when(c0 + 2 < NCHUNK)
        def _():
            fire(c0 + 2, 0)

        compute(c0, 0)
        wait_buf(1)

        @pl.when(c0 + 3 < NCHUNK)
        def _():
            fire(c0 + 3, 1)

        compute(c0 + 1, 1)
        return 0

    lax.fori_loop(0, NCHUNK // 2, pair_body, 0)

    pltpu.sync_copy(outbuf, out_hbm.at[pl.ds(base, RPW)])


def _distmult_sc(h, r, t, ent, rel):
    mesh = plsc.VectorSubcoreMesh(core_axis_name="c", subcore_axis_name="s")
    k = functools.partial(
        pl.kernel,
        out_type=jax.ShapeDtypeStruct((B,), jnp.float32),
        mesh=mesh,
        compiler_params=pltpu.CompilerParams(needs_layout_passes=False),
        scratch_types=[
            pltpu.VMEM((RPW,), jnp.int32),
            pltpu.VMEM((RPW,), jnp.int32),
            pltpu.VMEM((RPW,), jnp.int32),
            pltpu.VMEM((CHUNK, D), jnp.float32),
            pltpu.VMEM((CHUNK, D), jnp.float32),
            pltpu.VMEM((CHUNK, D), jnp.float32),
            pltpu.VMEM((CHUNK, D), jnp.float32),
            pltpu.VMEM((CHUNK, D), jnp.float32),
            pltpu.VMEM((CHUNK, D), jnp.float32),
            pltpu.VMEM((L * L,), jnp.float32),
            pltpu.VMEM((RPW,), jnp.float32),
            pltpu.SemaphoreType.DMA,
            pltpu.SemaphoreType.DMA,
        ],
    )(_body)
    return k(h, r, t, ent, rel)


def kernel(h, r, t, entity_embeddings, relation_embeddings):
    h = jnp.asarray(h, jnp.int32)
    r = jnp.asarray(r, jnp.int32)
    t = jnp.asarray(t, jnp.int32)
    return _distmult_sc(h, r, t, entity_embeddings, relation_embeddings)
